# baseline (speedup 1.0000x reference)
import functools

import jax
import jax.numpy as jnp
from jax import lax
from jax.experimental import pallas as pl
from jax.experimental.pallas import tpu as pltpu
from jax.experimental.pallas import tpu_sc as plsc

_BT = 1024


def _rtne_bf16_bits(u):
    return u + jnp.uint32(0x7FFF) + ((u >> 16) & jnp.uint32(1))


_TS = 256


def _pack_bf16_pairs(m):
    half = m.shape[1] // 2
    u_lo = _rtne_bf16_bits(lax.bitcast_convert_type(m[:, :half], jnp.uint32))
    u_hi = _rtne_bf16_bits(lax.bitcast_convert_type(m[:, half:], jnp.uint32))
    packed = (u_hi & jnp.uint32(0xFFFF0000)) | (u_lo >> 16)
    return lax.bitcast_convert_type(packed, jnp.int32)


def _s1_prep_v(wi_ref, cb_ref, v_ref, cn2_ref):
    dd = wi_ref.shape[0]
    wbf = wi_ref[...].astype(jnp.bfloat16)
    cbbf = cb_ref[...].astype(jnp.bfloat16)
    v_ref[:dd, :] = wbf
    g = lax.dot_general(cbbf, wbf, (((1,), (0,)), ((), ())),
                        preferred_element_type=jnp.float32)
    v_ref[dd:, :] = (-2.0 * g).astype(jnp.bfloat16)
    cbf = cb_ref[...]
    ones = jnp.ones((1, dd), jnp.float32)
    cn2_ref[...] = lax.dot_general(ones, cbf * cbf, (((1,), (1,)), ((), ())),
                                   preferred_element_type=jnp.float32) + 2.0
    return cbbf


def _s1_common(i, h_ref, v_ref, cn2_ref, am_ref, idx_ref, acc_ref):
    dd = h_ref.shape[1]
    nc = cn2_ref.shape[1]
    cn2 = cn2_ref[...]
    parts = []
    for t in range(_BT // _TS):
        h = h_ref[t * _TS:(t + 1) * _TS, :]
        zs = lax.dot_general(h.astype(jnp.bfloat16), v_ref[...],
                             (((1,), (1,)), ((), ())),
                             preferred_element_type=jnp.float32)
        z = zs[:, :dd]
        dp = cn2 + zs[:, dd:]
        cols = lax.broadcasted_iota(jnp.int32, dp.shape, 1)
        pk = (lax.bitcast_convert_type(dp, jnp.int32)
              & jnp.int32(~0x3FF)) | cols
        pmin = jnp.min(pk, axis=1)
        am = am_ref[pl.ds(t * _TS, _TS)]
        pad_row = nc + ((i * _BT + t * _TS + lax.iota(jnp.int32, _TS))
                        & (nc - 1))
        idx_ref[pl.ds(t * _TS, _TS)] = jnp.where(am > 0, pmin & 0x3FF,
                                                 pad_row)
        dmin = lax.bitcast_convert_type(pmin & jnp.int32(~0x3FF),
                                        jnp.float32) - 2.0
        zn = lax.dot_general(z * z, jnp.ones((1, dd), jnp.float32),
                             (((1,), (1,)), ((), ())),
                             preferred_element_type=jnp.float32)
        parts.append(jnp.sum(zn, axis=(0, 1), keepdims=True)
                     + jnp.sum(dmin))
    part = sum(parts[1:], parts[0])

    @pl.when(i == 0)
    def _init():
        acc_ref[...] = part

    @pl.when(i != 0)
    def _accum():
        acc_ref[...] += part


def _s1a_body(h_ref, wi_ref, cb_ref, wo_ref, am_ref,
              idx_ref, acc_ref, cbw_ref, v_ref, cn2_ref):
    i = pl.program_id(0)

    @pl.when(i == 0)
    def _prep():
        cbbf = _s1_prep_v(wi_ref, cb_ref, v_ref, cn2_ref)
        m = lax.dot_general(cbbf, wo_ref[...].astype(jnp.bfloat16),
                            (((1,), (1,)), ((), ())),
                            preferred_element_type=jnp.float32)
        cbw_ref[:m.shape[0], :] = _pack_bf16_pairs(m)
        cbw_ref[m.shape[0]:, :] = jnp.zeros(
            (cbw_ref.shape[0] - m.shape[0], m.shape[1] // 2), jnp.int32)

    _s1_common(i, h_ref, v_ref, cn2_ref, am_ref, idx_ref, acc_ref)


def _s1b_body(h_ref, wi_ref, cb_ref, am_ref,
              idx_ref, acc_ref, v_ref, cn2_ref):
    i = pl.program_id(0)

    @pl.when(i == 0)
    def _prep():
        _s1_prep_v(wi_ref, cb_ref, v_ref, cn2_ref)

    _s1_common(i, h_ref, v_ref, cn2_ref, am_ref, idx_ref, acc_ref)


def _s3_alias_body(prev_ref, out_ref, h_ref, g_ref, b_ref, o_ref):
    del prev_ref
    _s3_body(out_ref, h_ref, g_ref, b_ref, o_ref)


def _s3_body(out_ref, h_ref, g_ref, b_ref, o_ref):
    u = lax.bitcast_convert_type(out_ref[...], jnp.uint32)
    f_lo = lax.bitcast_convert_type(u << 16, jnp.float32)
    f_hi = lax.bitcast_convert_type(u & jnp.uint32(0xFFFF0000), jnp.float32)
    half = u.shape[1]
    dim = 2 * half
    x0 = h_ref[:, :half] + f_lo
    x1 = h_ref[:, half:] + f_hi
    mu = (jnp.sum(x0, axis=1, keepdims=True)
          + jnp.sum(x1, axis=1, keepdims=True)) * (1.0 / dim)
    xc0 = x0 - mu
    xc1 = x1 - mu
    var = (jnp.sum(xc0 * xc0, axis=1, keepdims=True)
           + jnp.sum(xc1 * xc1, axis=1, keepdims=True)) * (1.0 / dim)
    r = lax.rsqrt(var + 1e-5)
    o_ref[:, :half] = xc0 * r * g_ref[:, :half] + b_ref[:, :half]
    o_ref[:, half:] = xc1 * r * g_ref[:, half:] + b_ref[:, half:]


def _make_sc_gather(num_tokens, dim):
    info = plsc.get_sparse_core_info()
    nc, ns = info.num_cores, info.num_subcores
    nw = nc * ns
    b_per_w = num_tokens // nw
    ch = 128
    n_ch = b_per_w // ch
    mesh = plsc.VectorSubcoreMesh(core_axis_name="c", subcore_axis_name="s")

    @functools.partial(
        pl.kernel, mesh=mesh,
        out_type=jax.ShapeDtypeStruct((num_tokens, dim), jnp.int32),
        scratch_types=[
            pltpu.VMEM((ch,), jnp.int32),
            pltpu.VMEM((ch, dim), jnp.int32),
            pltpu.SemaphoreType.DMA,
        ],
    )
    def gather(table_hbm, idx_hbm, out_hbm, idx_v, rows_v, sem):
        wid = lax.axis_index("s") * nc + lax.axis_index("c")
        base = wid * b_per_w
        for c in range(n_ch):
            off = base + c * ch
            pltpu.sync_copy(idx_hbm.at[pl.ds(off, ch)], idx_v)
            pltpu.async_copy(table_hbm.at[idx_v], rows_v, sem).wait()
            pltpu.sync_copy(rows_v, out_hbm.at[pl.ds(off, ch)])

    return gather


def kernel(hidden, codebook, W_in, W_out, ln_g, ln_b, active_mask):
    d = hidden.shape[-1]
    n = hidden.shape[0] * hidden.shape[1]
    c = codebook.shape[0]
    h2 = hidden.reshape(n, d)
    nblk = n // _BT

    am_i32 = active_mask.reshape(n).astype(jnp.int32)
    nh = n // 2
    nblk = nh // _BT
    gather = _make_sc_gather(nh, d // 2)
    scratch = [
        pltpu.VMEM((d + c, d), jnp.bfloat16),
        pltpu.VMEM((1, c), jnp.float32),
    ]

    def s1a():
        return pl.pallas_call(
            _s1a_body,
            grid=(nblk,),
            in_specs=[
                pl.BlockSpec((_BT, d), lambda i: (i, 0)),
                pl.BlockSpec((d, d), lambda i: (0, 0)),
                pl.BlockSpec((c, d), lambda i: (0, 0)),
                pl.BlockSpec((d, d), lambda i: (0, 0)),
                pl.BlockSpec((_BT,), lambda i: (i,)),
            ],
            out_specs=[
                pl.BlockSpec((_BT,), lambda i: (i,)),
                pl.BlockSpec((1, 1), lambda i: (0, 0)),
                pl.BlockSpec((2 * c, d // 2), lambda i: (0, 0)),
            ],
            out_shape=[
                jax.ShapeDtypeStruct((nh,), jnp.int32),
                jax.ShapeDtypeStruct((1, 1), jnp.float32),
                jax.ShapeDtypeStruct((2 * c, d // 2), jnp.int32),
            ],
            scratch_shapes=scratch,
        )(h2, W_in, codebook, W_out, am_i32)

    def s1b():
        return pl.pallas_call(
            _s1b_body,
            grid=(nblk,),
            in_specs=[
                pl.BlockSpec((_BT, d), lambda i: (i + nblk, 0)),
                pl.BlockSpec((d, d), lambda i: (0, 0)),
                pl.BlockSpec((c, d), lambda i: (0, 0)),
                pl.BlockSpec((_BT,), lambda i: (i + nblk,)),
            ],
            out_specs=[
                pl.BlockSpec((_BT,), lambda i: (i,)),
                pl.BlockSpec((1, 1), lambda i: (0, 0)),
            ],
            out_shape=[
                jax.ShapeDtypeStruct((nh,), jnp.int32),
                jax.ShapeDtypeStruct((1, 1), jnp.float32),
            ],
            scratch_shapes=scratch,
        )(h2, W_in, codebook, am_i32)

    def s3_first(rows_half):
        return pl.pallas_call(
            _s3_body,
            grid=(nblk,),
            in_specs=[
                pl.BlockSpec((_BT, d // 2), lambda i: (i, 0)),
                pl.BlockSpec((_BT, d), lambda i: (i, 0)),
                pl.BlockSpec((1, d), lambda i: (0, 0)),
                pl.BlockSpec((1, d), lambda i: (0, 0)),
            ],
            out_specs=pl.BlockSpec((_BT, d), lambda i: (i, 0)),
            out_shape=jax.ShapeDtypeStruct((n, d), jnp.float32),
        )(rows_half, h2, ln_g.reshape(1, d), ln_b.reshape(1, d))

    def s3_second(prev, rows_half):
        return pl.pallas_call(
            _s3_alias_body,
            grid=(nblk,),
            in_specs=[
                pl.BlockSpec((8, 128), lambda i: (0, 0)),
                pl.BlockSpec((_BT, d // 2), lambda i: (i, 0)),
                pl.BlockSpec((_BT, d), lambda i: (i + nblk, 0)),
                pl.BlockSpec((1, d), lambda i: (0, 0)),
                pl.BlockSpec((1, d), lambda i: (0, 0)),
            ],
            out_specs=pl.BlockSpec((_BT, d), lambda i: (i + nblk, 0)),
            out_shape=jax.ShapeDtypeStruct((n, d), jnp.float32),
            input_output_aliases={0: 0},
        )(prev, rows_half, h2, ln_g.reshape(1, d), ln_b.reshape(1, d))

    idx_a, acc_a, cbw = s1a()
    rows_a = gather(cbw, idx_a)
    idx_b, acc_b = s1b()
    rows_b = gather(cbw, idx_b)
    hc_a = s3_first(rows_a)
    h_comm = s3_second(hc_a, rows_b)

    vq_loss = (1.0 + 0.25) * (acc_a[0, 0] + acc_b[0, 0]) / (n * d)
    return h_comm.reshape(hidden.shape), vq_loss


# speedup vs baseline: 1.0113x; 1.0113x over previous
import functools

import jax
import jax.numpy as jnp
from jax import lax
from jax.experimental import pallas as pl
from jax.experimental.pallas import tpu as pltpu
from jax.experimental.pallas import tpu_sc as plsc

_BT = 1024


def _rtne_bf16_bits(u):
    return u + jnp.uint32(0x7FFF) + ((u >> 16) & jnp.uint32(1))


_TS = 256


def _pack_bf16_pairs(m):
    half = m.shape[1] // 2
    u_lo = _rtne_bf16_bits(lax.bitcast_convert_type(m[:, :half], jnp.uint32))
    u_hi = _rtne_bf16_bits(lax.bitcast_convert_type(m[:, half:], jnp.uint32))
    packed = (u_hi & jnp.uint32(0xFFFF0000)) | (u_lo >> 16)
    return lax.bitcast_convert_type(packed, jnp.int32)


def _s1_prep_v(wi_ref, cb_ref, v_ref, cn2_ref):
    dd = wi_ref.shape[0]
    wbf = wi_ref[...].astype(jnp.bfloat16)
    cbbf = cb_ref[...].astype(jnp.bfloat16)
    v_ref[:dd, :] = wbf
    g = lax.dot_general(cbbf, wbf, (((1,), (0,)), ((), ())),
                        preferred_element_type=jnp.float32)
    v_ref[dd:, :] = (-2.0 * g).astype(jnp.bfloat16)
    cbf = cb_ref[...]
    ones = jnp.ones((1, dd), jnp.float32)
    cn2_ref[...] = lax.dot_general(ones, cbf * cbf, (((1,), (1,)), ((), ())),
                                   preferred_element_type=jnp.float32) + 2.0
    return cbbf


def _s1_common(i, h_ref, v_ref, cn2_ref, am_ref, idx_ref, acc_ref):
    dd = h_ref.shape[1]
    nc = cn2_ref.shape[1]
    cn2 = cn2_ref[...]
    parts = []
    zs_all = []
    for t in range(_BT // _TS):
        h = h_ref[t * _TS:(t + 1) * _TS, :]
        zs_all.append(lax.dot_general(h.astype(jnp.bfloat16), v_ref[...],
                                      (((1,), (1,)), ((), ())),
                                      preferred_element_type=jnp.float32))
    for t in range(_BT // _TS):
        zs = zs_all[t]
        z = zs[:, :dd]
        dp = cn2 + zs[:, dd:]
        cols = lax.broadcasted_iota(jnp.int32, dp.shape, 1)
        pk = (lax.bitcast_convert_type(dp, jnp.int32)
              & jnp.int32(~0x3FF)) | cols
        pmin = jnp.min(pk, axis=1)
        am = am_ref[pl.ds(t * _TS, _TS)]
        pad_row = nc + ((i * _BT + t * _TS + lax.iota(jnp.int32, _TS))
                        & (nc - 1))
        idx_ref[pl.ds(t * _TS, _TS)] = jnp.where(am > 0, pmin & 0x3FF,
                                                 pad_row)
        dmin = lax.bitcast_convert_type(pmin & jnp.int32(~0x3FF),
                                        jnp.float32) - 2.0
        zn = lax.dot_general(z * z, jnp.ones((1, dd), jnp.float32),
                             (((1,), (1,)), ((), ())),
                             preferred_element_type=jnp.float32)
        parts.append(jnp.sum(zn, axis=(0, 1), keepdims=True)
                     + jnp.sum(dmin))
    part = sum(parts[1:], parts[0])

    @pl.when(i == 0)
    def _init():
        acc_ref[...] = part

    @pl.when(i != 0)
    def _accum():
        acc_ref[...] += part


def _s1a_body(h_ref, wi_ref, cb_ref, wo_ref, am_ref,
              idx_ref, acc_ref, cbw_ref, v_ref, cn2_ref):
    i = pl.program_id(0)

    @pl.when(i == 0)
    def _prep():
        cbbf = _s1_prep_v(wi_ref, cb_ref, v_ref, cn2_ref)
        m = lax.dot_general(cbbf, wo_ref[...].astype(jnp.bfloat16),
                            (((1,), (1,)), ((), ())),
                            preferred_element_type=jnp.float32)
        cbw_ref[:m.shape[0], :] = _pack_bf16_pairs(m)
        cbw_ref[m.shape[0]:, :] = jnp.zeros(
            (cbw_ref.shape[0] - m.shape[0], m.shape[1] // 2), jnp.int32)

    _s1_common(i, h_ref, v_ref, cn2_ref, am_ref, idx_ref, acc_ref)


def _s1b_body(h_ref, wi_ref, cb_ref, am_ref,
              idx_ref, acc_ref, v_ref, cn2_ref):
    i = pl.program_id(0)

    @pl.when(i == 0)
    def _prep():
        _s1_prep_v(wi_ref, cb_ref, v_ref, cn2_ref)

    _s1_common(i, h_ref, v_ref, cn2_ref, am_ref, idx_ref, acc_ref)


def _s3_alias_body(prev_ref, out_ref, h_ref, g_ref, b_ref, o_ref):
    del prev_ref
    _s3_body(out_ref, h_ref, g_ref, b_ref, o_ref)


def _s3_body(out_ref, h_ref, g_ref, b_ref, o_ref):
    u = lax.bitcast_convert_type(out_ref[...], jnp.uint32)
    f_lo = lax.bitcast_convert_type(u << 16, jnp.float32)
    f_hi = lax.bitcast_convert_type(u & jnp.uint32(0xFFFF0000), jnp.float32)
    half = u.shape[1]
    dim = 2 * half
    x0 = h_ref[:, :half] + f_lo
    x1 = h_ref[:, half:] + f_hi
    mu = (jnp.sum(x0, axis=1, keepdims=True)
          + jnp.sum(x1, axis=1, keepdims=True)) * (1.0 / dim)
    xc0 = x0 - mu
    xc1 = x1 - mu
    var = (jnp.sum(xc0 * xc0, axis=1, keepdims=True)
           + jnp.sum(xc1 * xc1, axis=1, keepdims=True)) * (1.0 / dim)
    r = lax.rsqrt(var + 1e-5)
    o_ref[:, :half] = xc0 * r * g_ref[:, :half] + b_ref[:, :half]
    o_ref[:, half:] = xc1 * r * g_ref[:, half:] + b_ref[:, half:]


def _make_sc_gather(num_tokens, dim):
    info = plsc.get_sparse_core_info()
    nc, ns = info.num_cores, info.num_subcores
    nw = nc * ns
    b_per_w = num_tokens // nw
    ch = 128
    n_ch = b_per_w // ch
    mesh = plsc.VectorSubcoreMesh(core_axis_name="c", subcore_axis_name="s")

    @functools.partial(
        pl.kernel, mesh=mesh,
        out_type=jax.ShapeDtypeStruct((num_tokens, dim), jnp.int32),
        scratch_types=[
            pltpu.VMEM((ch,), jnp.int32),
            pltpu.VMEM((ch, dim), jnp.int32),
            pltpu.SemaphoreType.DMA,
        ],
    )
    def gather(table_hbm, idx_hbm, out_hbm, idx_v, rows_v, sem):
        wid = lax.axis_index("s") * nc + lax.axis_index("c")
        base = wid * b_per_w
        for c in range(n_ch):
            off = base + c * ch
            pltpu.sync_copy(idx_hbm.at[pl.ds(off, ch)], idx_v)
            pltpu.async_copy(table_hbm.at[idx_v], rows_v, sem).wait()
            pltpu.sync_copy(rows_v, out_hbm.at[pl.ds(off, ch)])

    return gather


def kernel(hidden, codebook, W_in, W_out, ln_g, ln_b, active_mask):
    d = hidden.shape[-1]
    n = hidden.shape[0] * hidden.shape[1]
    c = codebook.shape[0]
    h2 = hidden.reshape(n, d)
    nblk = n // _BT

    am_i32 = active_mask.reshape(n).astype(jnp.int32)
    nh = n // 2
    nblk = nh // _BT
    gather = _make_sc_gather(nh, d // 2)
    scratch = [
        pltpu.VMEM((d + c, d), jnp.bfloat16),
        pltpu.VMEM((1, c), jnp.float32),
    ]

    def s1a():
        return pl.pallas_call(
            _s1a_body,
            grid=(nblk,),
            in_specs=[
                pl.BlockSpec((_BT, d), lambda i: (i, 0)),
                pl.BlockSpec((d, d), lambda i: (0, 0)),
                pl.BlockSpec((c, d), lambda i: (0, 0)),
                pl.BlockSpec((d, d), lambda i: (0, 0)),
                pl.BlockSpec((_BT,), lambda i: (i,)),
            ],
            out_specs=[
                pl.BlockSpec((_BT,), lambda i: (i,)),
                pl.BlockSpec((1, 1), lambda i: (0, 0)),
                pl.BlockSpec((2 * c, d // 2), lambda i: (0, 0)),
            ],
            out_shape=[
                jax.ShapeDtypeStruct((nh,), jnp.int32),
                jax.ShapeDtypeStruct((1, 1), jnp.float32),
                jax.ShapeDtypeStruct((2 * c, d // 2), jnp.int32),
            ],
            scratch_shapes=scratch,
        )(h2, W_in, codebook, W_out, am_i32)

    def s1b():
        return pl.pallas_call(
            _s1b_body,
            grid=(nblk,),
            in_specs=[
                pl.BlockSpec((_BT, d), lambda i: (i + nblk, 0)),
                pl.BlockSpec((d, d), lambda i: (0, 0)),
                pl.BlockSpec((c, d), lambda i: (0, 0)),
                pl.BlockSpec((_BT,), lambda i: (i + nblk,)),
            ],
            out_specs=[
                pl.BlockSpec((_BT,), lambda i: (i,)),
                pl.BlockSpec((1, 1), lambda i: (0, 0)),
            ],
            out_shape=[
                jax.ShapeDtypeStruct((nh,), jnp.int32),
                jax.ShapeDtypeStruct((1, 1), jnp.float32),
            ],
            scratch_shapes=scratch,
        )(h2, W_in, codebook, am_i32)

    def s3_first(rows_half):
        return pl.pallas_call(
            _s3_body,
            grid=(nblk,),
            in_specs=[
                pl.BlockSpec((_BT, d // 2), lambda i: (i, 0)),
                pl.BlockSpec((_BT, d), lambda i: (i, 0)),
                pl.BlockSpec((1, d), lambda i: (0, 0)),
                pl.BlockSpec((1, d), lambda i: (0, 0)),
            ],
            out_specs=pl.BlockSpec((_BT, d), lambda i: (i, 0)),
            out_shape=jax.ShapeDtypeStruct((n, d), jnp.float32),
        )(rows_half, h2, ln_g.reshape(1, d), ln_b.reshape(1, d))

    def s3_second(prev, rows_half):
        return pl.pallas_call(
            _s3_alias_body,
            grid=(nblk,),
            in_specs=[
                pl.BlockSpec((8, 128), lambda i: (0, 0)),
                pl.BlockSpec((_BT, d // 2), lambda i: (i, 0)),
                pl.BlockSpec((_BT, d), lambda i: (i + nblk, 0)),
                pl.BlockSpec((1, d), lambda i: (0, 0)),
                pl.BlockSpec((1, d), lambda i: (0, 0)),
            ],
            out_specs=pl.BlockSpec((_BT, d), lambda i: (i + nblk, 0)),
            out_shape=jax.ShapeDtypeStruct((n, d), jnp.float32),
            input_output_aliases={0: 0},
        )(prev, rows_half, h2, ln_g.reshape(1, d), ln_b.reshape(1, d))

    idx_a, acc_a, cbw = s1a()
    rows_a = gather(cbw, idx_a)
    idx_b, acc_b = s1b()
    rows_b = gather(cbw, idx_b)
    hc_a = s3_first(rows_a)
    h_comm = s3_second(hc_a, rows_b)

    vq_loss = (1.0 + 0.25) * (acc_a[0, 0] + acc_b[0, 0]) / (n * d)
    return h_comm.reshape(hidden.shape), vq_loss
